# baseline (speedup 1.0000x reference)
import jax
import jax.numpy as jnp
from jax import lax
from jax.experimental import pallas as pl
from jax.experimental.pallas import tpu as pltpu
from jax.experimental.pallas import tpu_sc as plsc

V = 100000
F = 200000
NNZ = 700000

NC = 2
NS = 16
NW = NC * NS
LANES = 16

S = 2736
NSUB = 8
Q = NSUB * S
OVERLAP = NW * Q - NNZ

VP = 100096
CHK = VP // NS
CHK_LAST = V - (NS - 1) * CHK


def _sc_spmv_call(vflat, cpad, ind_l, val_l, ind_k, val_k):
    mesh = plsc.VectorSubcoreMesh(core_axis_name="c", subcore_axis_name="s",
                                  num_cores=NC, num_subcores=NS)

    def body(vflat_hbm, cpad_hbm, iL_hbm, vL_hbm, iK_hbm, vK_hbm,
             out_hbm,
             aLx, aLy, aLz, aKx, aKy, aKz, sx, sy, sz,
             ix, iy, iz, xb, yb, zb, c_v,
             col_v, row_v, val_v, gx, gy, gz,
             sem0, sem1, sem2):
        cid = lax.axis_index("c")
        sid = lax.axis_index("s")

        def zero_body(i, _):
            xb[pl.ds(i * LANES, LANES)] = jnp.zeros((LANES,), jnp.float32)
            return 0
        lax.fori_loop(0, CHK // LANES, zero_body, 0)
        for acc in (aLx, aLy, aLz, aKx, aKy, aKz):
            pltpu.sync_copy(xb, acc.at[pl.ds(sid * CHK, CHK)])

        n = jnp.where(sid == NS - 1, CHK_LAST, CHK)
        base0 = sid * CHK
        pltpu.sync_copy(cpad_hbm, c_v)
        iota = lax.iota(jnp.int32, LANES)
        cxv = c_v[pl.ds(0, LANES)]
        cyv = c_v[pl.ds(LANES, LANES)]
        czv = c_v[pl.ds(2 * LANES, LANES)]

        def idx_body(i, _):
            sl = pl.ds(i * LANES, LANES)
            v3 = jnp.minimum(base0 + i * LANES + iota, V - 1) * 3
            ix[sl] = v3
            iy[sl] = v3 + 1
            iz[sl] = v3 + 2
            return 0
        lax.fori_loop(0, CHK // LANES, idx_body, 0)
        d0 = pltpu.async_copy(vflat_hbm.at[ix], xb, sem0)
        d1 = pltpu.async_copy(vflat_hbm.at[iy], yb, sem1)
        d2 = pltpu.async_copy(vflat_hbm.at[iz], zb, sem2)
        d0.wait()
        d1.wait()
        d2.wait()

        def cadd_body(i, _):
            sl = pl.ds(i * LANES, LANES)
            xb[sl] = xb[sl] + cxv
            yb[sl] = yb[sl] + cyv
            zb[sl] = zb[sl] + czv
            return 0
        lax.fori_loop(0, CHK // LANES, cadd_body, 0)
        for buf, ssrc in ((xb, sx), (yb, sy), (zb, sz)):
            pltpu.sync_copy(buf.at[pl.ds(0, n)],
                            ssrc.at[pl.ds(sid * CHK, n)])
        plsc.subcore_barrier()

        wid = cid * NS + sid
        last = jnp.logical_and(wid == NW - 1, True)

        def do_matrix(i_hbm, v_hbm, ax, ay, az):
            for j in range(NSUB):
                base = wid * Q + j * S
                if j == NSUB - 1:
                    base = jnp.where(wid == NW - 1, NNZ - S, base)
                pltpu.sync_copy(i_hbm.at[pl.ds(base, S)], row_v)
                pltpu.sync_copy(i_hbm.at[pl.ds(NNZ + base, S)], col_v)
                pltpu.sync_copy(v_hbm.at[pl.ds(base, S)], val_v)
                if j == NSUB - 1:
                    @pl.when(last)
                    def _mask_tail():
                        def zv(i, _):
                            val_v[pl.ds(i * LANES, LANES)] = (
                                jnp.zeros((LANES,), jnp.float32))
                            return 0
                        lax.fori_loop(0, OVERLAP // LANES, zv, 0)
                d0 = pltpu.async_copy(sx.at[col_v], gx, sem0)
                d1 = pltpu.async_copy(sy.at[col_v], gy, sem1)
                d2 = pltpu.async_copy(sz.at[col_v], gz, sem2)
                d0.wait()
                d1.wait()
                d2.wait()

                def mul_body(i, _):
                    sl = pl.ds(i * LANES, LANES)
                    w = val_v[sl]
                    gx[sl] = gx[sl] * w
                    gy[sl] = gy[sl] * w
                    gz[sl] = gz[sl] * w
                    return 0
                lax.fori_loop(0, S // LANES, mul_body, 0)

                pltpu.sync_copy(gx, ax.at[row_v], add=True)
                pltpu.sync_copy(gy, ay.at[row_v], add=True)
                pltpu.sync_copy(gz, az.at[row_v], add=True)

        do_matrix(iL_hbm, vL_hbm, aLx, aLy, aLz)
        do_matrix(iK_hbm, vK_hbm, aKx, aKy, aKz)

        plsc.subcore_barrier()

        for j, acc in enumerate((aLx, aLy, aLz, aKx, aKy, aKz)):
            off = (cid * 6 + j) * VP + sid * CHK
            pltpu.sync_copy(acc.at[pl.ds(sid * CHK, CHK)], xb)
            pltpu.sync_copy(xb, out_hbm.at[pl.ds(off, CHK)])

    kfn = pl.kernel(
        body,
        out_type=jax.ShapeDtypeStruct((NC * 6 * VP,), jnp.float32),
        mesh=mesh,
        scratch_types=[
            pltpu.VMEM_SHARED((VP,), jnp.float32),
            pltpu.VMEM_SHARED((VP,), jnp.float32),
            pltpu.VMEM_SHARED((VP,), jnp.float32),
            pltpu.VMEM_SHARED((VP,), jnp.float32),
            pltpu.VMEM_SHARED((VP,), jnp.float32),
            pltpu.VMEM_SHARED((VP,), jnp.float32),
            pltpu.VMEM_SHARED((VP,), jnp.float32),
            pltpu.VMEM_SHARED((VP,), jnp.float32),
            pltpu.VMEM_SHARED((VP,), jnp.float32),
            pltpu.VMEM((CHK,), jnp.int32),
            pltpu.VMEM((CHK,), jnp.int32),
            pltpu.VMEM((CHK,), jnp.int32),
            pltpu.VMEM((CHK,), jnp.float32),
            pltpu.VMEM((CHK,), jnp.float32),
            pltpu.VMEM((CHK,), jnp.float32),
            pltpu.VMEM((3 * LANES,), jnp.float32),
            pltpu.VMEM((S,), jnp.int32),
            pltpu.VMEM((S,), jnp.int32),
            pltpu.VMEM((S,), jnp.float32),
            pltpu.VMEM((S,), jnp.float32),
            pltpu.VMEM((S,), jnp.float32),
            pltpu.VMEM((S,), jnp.float32),
            pltpu.SemaphoreType.DMA,
            pltpu.SemaphoreType.DMA,
            pltpu.SemaphoreType.DMA,
        ],
    )
    return kfn(vflat, cpad, ind_l, val_l, ind_k, val_k)


def _tc_tile_call(vflat, cpat, fflat, one_f, one_i):
    BV = 3072
    BF = 6144
    grid = pl.cdiv(3 * V, BV)

    def body(vf_ref, cp_ref, ff_ref, onef_ref, onei_ref, vout_ref, fout_ref):
        vrow = (vf_ref[...] + cp_ref[...]) * onef_ref[0, 0]
        vout_ref[...] = jnp.broadcast_to(vrow[None], (4, BV))
        fout_ref[...] = jnp.broadcast_to(
            ff_ref[...][None] * onei_ref[0, 0], (4, BF))

    return pl.pallas_call(
        body,
        grid=(grid,),
        in_specs=[
            pl.BlockSpec((BV,), lambda i: (i,)),
            pl.BlockSpec((BV,), lambda i: (0,)),
            pl.BlockSpec((BF,), lambda i: (i,)),
            pl.BlockSpec(memory_space=pltpu.SMEM),
            pl.BlockSpec(memory_space=pltpu.SMEM),
        ],
        out_specs=[
            pl.BlockSpec((4, BV), lambda i: (0, i)),
            pl.BlockSpec((4, BF), lambda i: (0, i)),
        ],
        out_shape=[
            jax.ShapeDtypeStruct((4, 3 * V), jnp.float32),
            jax.ShapeDtypeStruct((4, 3 * F), jnp.int32),
        ],
    )(vflat, cpat, fflat, one_f, one_i)


def _tc_reduce_call(parts):
    def body(p_ref, out_ref):
        p = p_ref[0] + p_ref[1]
        lap = p[0:3] + jnp.float32(1e-12)
        norm = jnp.sqrt(jnp.sum(lap * lap, axis=0))
        kv = p[3:6]
        ksq = jnp.sum(kv * kv, axis=0)
        out_ref[0, 0] = jnp.sum(norm) / jnp.float32(V)
        out_ref[0, 1] = jnp.sum(ksq) / jnp.float32(V)

    return pl.pallas_call(
        body,
        out_specs=pl.BlockSpec(memory_space=pltpu.SMEM),
        out_shape=jax.ShapeDtypeStruct((1, 2), jnp.float32),
    )(parts)


def kernel(vertices, center, faces, L_indices, L_values, K_indices, K_values,
           total_num):
    one_i = jnp.asarray(total_num, dtype=jnp.int32) // 4
    one_f = one_i.astype(jnp.float32)
    one_i_s = jnp.reshape(one_i, (1, 1))
    one_f_s = jnp.reshape(one_f, (1, 1))

    vflat = vertices.reshape(3 * V)
    fflat = faces.reshape(3 * F)
    c3 = center.reshape(3)
    cpad = jnp.repeat(c3, LANES)
    cpat = jnp.tile(c3, 1024)

    parts = _sc_spmv_call(vflat, cpad, L_indices.reshape(2 * NNZ), L_values,
                          K_indices.reshape(2 * NNZ), K_values)
    vo_flat, fo_flat = _tc_tile_call(vflat, cpat, fflat, one_f_s, one_i_s)
    losses = _tc_reduce_call(parts.reshape(NC, 6, VP))

    verts_out = vo_flat.reshape(4, V, 3)
    faces_out = fo_flat.reshape(4, F, 3)
    laplacian_loss = losses[0, 0]
    hexagon_loss = losses[0, 1]
    zero = jnp.float32(0.0)
    return (verts_out, faces_out, laplacian_loss, hexagon_loss, zero, zero)


# speedup vs baseline: 1.0142x; 1.0142x over previous
import jax
import jax.numpy as jnp
from jax import lax
from jax.experimental import pallas as pl
from jax.experimental.pallas import tpu as pltpu
from jax.experimental.pallas import tpu_sc as plsc

V = 100000
F = 200000
NNZ = 700000

NC = 2
NS = 16
NW = NC * NS
LANES = 16

S = 2736
NSUB = 8
Q = NSUB * S
OVERLAP = NW * Q - NNZ

VP = 100096
CHK = VP // NS


def _sc_spmv_call(vflat, cpat48, ind_l, val_l, ind_k, val_k):
    mesh = plsc.VectorSubcoreMesh(core_axis_name="c", subcore_axis_name="s",
                                  num_cores=NC, num_subcores=NS)

    def body(vflat_hbm, cpat_hbm, iL_hbm, vL_hbm, iK_hbm, vK_hbm,
             out_hbm,
             aLx, aLy, aLz, aKx, aKy, aKz, sv3,
             vbuf, xb, cp_v,
             col_v, row_v, val_v, ic0, ic1, ic2, gx, gy, gz,
             sem0, sem1, sem2):
        cid = lax.axis_index("c")
        sid = lax.axis_index("s")

        def zero_body(i, _):
            xb[pl.ds(i * LANES, LANES)] = jnp.zeros((LANES,), jnp.float32)
            return 0
        lax.fori_loop(0, CHK // LANES, zero_body, 0)
        for acc in (aLx, aLy, aLz, aKx, aKy, aKz):
            pltpu.sync_copy(xb, acc.at[pl.ds(sid * CHK, CHK)])

        base0 = jnp.minimum(sid * CHK, V - CHK)
        pltpu.sync_copy(vflat_hbm.at[pl.ds(base0 * 3, 3 * CHK)], vbuf)
        pltpu.sync_copy(cpat_hbm, cp_v)
        q0 = cp_v[pl.ds(0, LANES)]
        q1 = cp_v[pl.ds(LANES, LANES)]
        q2 = cp_v[pl.ds(2 * LANES, LANES)]

        def cadd_body(g, _):
            b = g * (3 * LANES)
            vbuf[pl.ds(b, LANES)] = vbuf[pl.ds(b, LANES)] + q0
            vbuf[pl.ds(b + LANES, LANES)] = (
                vbuf[pl.ds(b + LANES, LANES)] + q1)
            vbuf[pl.ds(b + 2 * LANES, LANES)] = (
                vbuf[pl.ds(b + 2 * LANES, LANES)] + q2)
            return 0
        lax.fori_loop(0, CHK // LANES, cadd_body, 0)
        pltpu.sync_copy(vbuf, sv3.at[pl.ds(base0 * 3, 3 * CHK)])
        plsc.subcore_barrier()

        wid = cid * NS + sid
        is_last = wid == NW - 1

        def do_matrix(i_hbm, v_hbm, ax, ay, az):
            for j in range(NSUB):
                base = wid * Q + j * S
                if j == NSUB - 1:
                    base = jnp.where(is_last, NNZ - S, base)
                pltpu.sync_copy(i_hbm.at[pl.ds(base, S)], row_v)
                pltpu.sync_copy(i_hbm.at[pl.ds(NNZ + base, S)], col_v)
                pltpu.sync_copy(v_hbm.at[pl.ds(base, S)], val_v)
                if j == NSUB - 1:
                    @pl.when(is_last)
                    def _mask_tail():
                        def zv(i, _):
                            val_v[pl.ds(i * LANES, LANES)] = (
                                jnp.zeros((LANES,), jnp.float32))
                            return 0
                        lax.fori_loop(0, OVERLAP // LANES, zv, 0)

                def idx_body(i, _):
                    sl = pl.ds(i * LANES, LANES)
                    c3 = col_v[sl] * 3
                    ic0[sl] = c3
                    ic1[sl] = c3 + 1
                    ic2[sl] = c3 + 2
                    return 0
                lax.fori_loop(0, S // LANES, idx_body, 0)
                d0 = pltpu.async_copy(sv3.at[ic0], gx, sem0)
                d1 = pltpu.async_copy(sv3.at[ic1], gy, sem1)
                d2 = pltpu.async_copy(sv3.at[ic2], gz, sem2)
                d0.wait()
                d1.wait()
                d2.wait()

                def mul_body(i, _):
                    sl = pl.ds(i * LANES, LANES)
                    w = val_v[sl]
                    gx[sl] = gx[sl] * w
                    gy[sl] = gy[sl] * w
                    gz[sl] = gz[sl] * w
                    return 0
                lax.fori_loop(0, S // LANES, mul_body, 0)

                pltpu.sync_copy(gx, ax.at[row_v], add=True)
                pltpu.sync_copy(gy, ay.at[row_v], add=True)
                pltpu.sync_copy(gz, az.at[row_v], add=True)

        do_matrix(iL_hbm, vL_hbm, aLx, aLy, aLz)
        do_matrix(iK_hbm, vK_hbm, aKx, aKy, aKz)

        plsc.subcore_barrier()

        for j, acc in enumerate((aLx, aLy, aLz, aKx, aKy, aKz)):
            off = (cid * 6 + j) * VP + sid * CHK
            pltpu.sync_copy(acc.at[pl.ds(sid * CHK, CHK)], xb)
            pltpu.sync_copy(xb, out_hbm.at[pl.ds(off, CHK)])

    kfn = pl.kernel(
        body,
        out_type=jax.ShapeDtypeStruct((NC * 6 * VP,), jnp.float32),
        mesh=mesh,
        scratch_types=[
            pltpu.VMEM_SHARED((VP,), jnp.float32),
            pltpu.VMEM_SHARED((VP,), jnp.float32),
            pltpu.VMEM_SHARED((VP,), jnp.float32),
            pltpu.VMEM_SHARED((VP,), jnp.float32),
            pltpu.VMEM_SHARED((VP,), jnp.float32),
            pltpu.VMEM_SHARED((VP,), jnp.float32),
            pltpu.VMEM_SHARED((3 * VP,), jnp.float32),
            pltpu.VMEM((3 * CHK,), jnp.float32),
            pltpu.VMEM((CHK,), jnp.float32),
            pltpu.VMEM((3 * LANES,), jnp.float32),
            pltpu.VMEM((S,), jnp.int32),
            pltpu.VMEM((S,), jnp.int32),
            pltpu.VMEM((S,), jnp.float32),
            pltpu.VMEM((S,), jnp.int32),
            pltpu.VMEM((S,), jnp.int32),
            pltpu.VMEM((S,), jnp.int32),
            pltpu.VMEM((S,), jnp.float32),
            pltpu.VMEM((S,), jnp.float32),
            pltpu.VMEM((S,), jnp.float32),
            pltpu.SemaphoreType.DMA,
            pltpu.SemaphoreType.DMA,
            pltpu.SemaphoreType.DMA,
        ],
    )
    return kfn(vflat, cpat48, ind_l, val_l, ind_k, val_k)


def _tc_tile_call(vflat, cpat, fflat, one_f, one_i):
    BV = 3072
    BF = 6144
    grid = pl.cdiv(3 * V, BV)

    def body(vf_ref, cp_ref, ff_ref, onef_ref, onei_ref, vout_ref, fout_ref):
        vrow = (vf_ref[...] + cp_ref[...]) * onef_ref[0, 0]
        vout_ref[...] = jnp.broadcast_to(vrow[None], (4, BV))
        fout_ref[...] = jnp.broadcast_to(
            ff_ref[...][None] * onei_ref[0, 0], (4, BF))

    return pl.pallas_call(
        body,
        grid=(grid,),
        in_specs=[
            pl.BlockSpec((BV,), lambda i: (i,)),
            pl.BlockSpec((BV,), lambda i: (0,)),
            pl.BlockSpec((BF,), lambda i: (i,)),
            pl.BlockSpec(memory_space=pltpu.SMEM),
            pl.BlockSpec(memory_space=pltpu.SMEM),
        ],
        out_specs=[
            pl.BlockSpec((4, BV), lambda i: (0, i)),
            pl.BlockSpec((4, BF), lambda i: (0, i)),
        ],
        out_shape=[
            jax.ShapeDtypeStruct((4, 3 * V), jnp.float32),
            jax.ShapeDtypeStruct((4, 3 * F), jnp.int32),
        ],
    )(vflat, cpat, fflat, one_f, one_i)


def _tc_reduce_call(parts):
    def body(p_ref, out_ref):
        p = p_ref[0] + p_ref[1]
        lap = p[0:3] + jnp.float32(1e-12)
        norm = jnp.sqrt(jnp.sum(lap * lap, axis=0))
        kv = p[3:6]
        ksq = jnp.sum(kv * kv, axis=0)
        out_ref[0, 0] = jnp.sum(norm) / jnp.float32(V)
        out_ref[0, 1] = jnp.sum(ksq) / jnp.float32(V)

    return pl.pallas_call(
        body,
        out_specs=pl.BlockSpec(memory_space=pltpu.SMEM),
        out_shape=jax.ShapeDtypeStruct((1, 2), jnp.float32),
    )(parts)


def kernel(vertices, center, faces, L_indices, L_values, K_indices, K_values,
           total_num):
    one_i = jnp.asarray(total_num, dtype=jnp.int32) // 4
    one_f = one_i.astype(jnp.float32)
    one_i_s = jnp.reshape(one_i, (1, 1))
    one_f_s = jnp.reshape(one_f, (1, 1))

    vflat = vertices.reshape(3 * V)
    fflat = faces.reshape(3 * F)
    c3 = center.reshape(3)
    cpat = jnp.tile(c3, 1024)
    cpat48 = cpat[:3 * LANES]

    parts = _sc_spmv_call(vflat, cpat48, L_indices.reshape(2 * NNZ), L_values,
                          K_indices.reshape(2 * NNZ), K_values)
    vo_flat, fo_flat = _tc_tile_call(vflat, cpat, fflat, one_f_s, one_i_s)
    losses = _tc_reduce_call(parts.reshape(NC, 6, VP))

    verts_out = vo_flat.reshape(4, V, 3)
    faces_out = fo_flat.reshape(4, F, 3)
    laplacian_loss = losses[0, 0]
    hexagon_loss = losses[0, 1]
    zero = jnp.float32(0.0)
    return (verts_out, faces_out, laplacian_loss, hexagon_loss, zero, zero)


# speedup vs baseline: 1.5871x; 1.5649x over previous
import jax
import jax.numpy as jnp
from jax import lax
from jax.experimental import pallas as pl
from jax.experimental.pallas import tpu as pltpu
from jax.experimental.pallas import tpu_sc as plsc

V = 100000
F = 200000
NNZ = 700000

NC = 2
NS = 16
NW = NC * NS
LANES = 16

S = 2736
NSUB = 8
Q = NSUB * S
OVERLAP = NW * Q - NNZ

VP = 100096
CHK = VP // NS


def _sc_spmv_call(vflat, cpat48, ind_l, val_l, ind_k, val_k):
    mesh = plsc.VectorSubcoreMesh(core_axis_name="c", subcore_axis_name="s",
                                  num_cores=NC, num_subcores=NS)

    def body(vflat_hbm, cpat_hbm, iL_hbm, vL_hbm, iK_hbm, vK_hbm,
             out_hbm,
             aLx, aLy, aLz, aKx, aKy, aKz, sv3,
             vbuf, xb, cp_v,
             col_v, row_v, val_v, ic0, ic1, ic2, gx, gy, gz,
             sem0, sem1, sem2):
        cid = lax.axis_index("c")
        sid = lax.axis_index("s")

        def zero_body(i, _):
            xb[pl.ds(i * LANES, LANES)] = jnp.zeros((LANES,), jnp.float32)
            return 0
        lax.fori_loop(0, CHK // LANES, zero_body, 0)
        for acc in (aLx, aLy, aLz, aKx, aKy, aKz):
            pltpu.sync_copy(xb, acc.at[pl.ds(sid * CHK, CHK)])

        base0 = jnp.minimum(sid * CHK, V - CHK)
        pltpu.sync_copy(vflat_hbm.at[pl.ds(base0 * 3, 3 * CHK)], vbuf)
        pltpu.sync_copy(cpat_hbm, cp_v)
        q0 = cp_v[pl.ds(0, LANES)]
        q1 = cp_v[pl.ds(LANES, LANES)]
        q2 = cp_v[pl.ds(2 * LANES, LANES)]

        def cadd_body(g, _):
            b = g * (3 * LANES)
            vbuf[pl.ds(b, LANES)] = vbuf[pl.ds(b, LANES)] + q0
            vbuf[pl.ds(b + LANES, LANES)] = (
                vbuf[pl.ds(b + LANES, LANES)] + q1)
            vbuf[pl.ds(b + 2 * LANES, LANES)] = (
                vbuf[pl.ds(b + 2 * LANES, LANES)] + q2)
            return 0
        lax.fori_loop(0, CHK // LANES, cadd_body, 0)
        pltpu.sync_copy(vbuf, sv3.at[pl.ds(base0 * 3, 3 * CHK)])
        plsc.subcore_barrier()

        wid = cid * NS + sid
        is_last = wid == NW - 1

        def do_matrix(i_hbm, v_hbm, ax, ay, az):
            for j in range(NSUB):
                base = wid * Q + j * S
                if j == NSUB - 1:
                    base = jnp.where(is_last, NNZ - S, base)
                pltpu.sync_copy(i_hbm.at[pl.ds(base, S)], row_v)
                pltpu.sync_copy(i_hbm.at[pl.ds(NNZ + base, S)], col_v)
                pltpu.sync_copy(v_hbm.at[pl.ds(base, S)], val_v)
                if j == NSUB - 1:
                    @pl.when(is_last)
                    def _mask_tail():
                        def zv(i, _):
                            val_v[pl.ds(i * LANES, LANES)] = (
                                jnp.zeros((LANES,), jnp.float32))
                            return 0
                        lax.fori_loop(0, OVERLAP // LANES, zv, 0)

                def idx_body(i, _):
                    sl = pl.ds(i * LANES, LANES)
                    c3 = col_v[sl] * 3
                    ic0[sl] = c3
                    ic1[sl] = c3 + 1
                    ic2[sl] = c3 + 2
                    return 0
                lax.fori_loop(0, S // LANES, idx_body, 0)
                d0 = pltpu.async_copy(sv3.at[ic0], gx, sem0)
                d1 = pltpu.async_copy(sv3.at[ic1], gy, sem1)
                d2 = pltpu.async_copy(sv3.at[ic2], gz, sem2)
                d0.wait()
                d1.wait()
                d2.wait()

                def mul_body(i, _):
                    sl = pl.ds(i * LANES, LANES)
                    w = val_v[sl]
                    gx[sl] = gx[sl] * w
                    gy[sl] = gy[sl] * w
                    gz[sl] = gz[sl] * w
                    return 0
                lax.fori_loop(0, S // LANES, mul_body, 0)

                pltpu.sync_copy(gx, ax.at[row_v], add=True)
                pltpu.sync_copy(gy, ay.at[row_v], add=True)
                pltpu.sync_copy(gz, az.at[row_v], add=True)

        do_matrix(iL_hbm, vL_hbm, aLx, aLy, aLz)
        do_matrix(iK_hbm, vK_hbm, aKx, aKy, aKz)

        plsc.subcore_barrier()

        for j, acc in enumerate((aLx, aLy, aLz, aKx, aKy, aKz)):
            off = (cid * 6 + j) * VP + sid * CHK
            pltpu.sync_copy(acc.at[pl.ds(sid * CHK, CHK)], xb)
            pltpu.sync_copy(xb, out_hbm.at[pl.ds(off, CHK)])

    kfn = pl.kernel(
        body,
        out_type=jax.ShapeDtypeStruct((NC * 6 * VP,), jnp.float32),
        mesh=mesh,
        scratch_types=[
            pltpu.VMEM_SHARED((VP,), jnp.float32),
            pltpu.VMEM_SHARED((VP,), jnp.float32),
            pltpu.VMEM_SHARED((VP,), jnp.float32),
            pltpu.VMEM_SHARED((VP,), jnp.float32),
            pltpu.VMEM_SHARED((VP,), jnp.float32),
            pltpu.VMEM_SHARED((VP,), jnp.float32),
            pltpu.VMEM_SHARED((3 * VP,), jnp.float32),
            pltpu.VMEM((3 * CHK,), jnp.float32),
            pltpu.VMEM((CHK,), jnp.float32),
            pltpu.VMEM((3 * LANES,), jnp.float32),
            pltpu.VMEM((S,), jnp.int32),
            pltpu.VMEM((S,), jnp.int32),
            pltpu.VMEM((S,), jnp.float32),
            pltpu.VMEM((S,), jnp.int32),
            pltpu.VMEM((S,), jnp.int32),
            pltpu.VMEM((S,), jnp.int32),
            pltpu.VMEM((S,), jnp.float32),
            pltpu.VMEM((S,), jnp.float32),
            pltpu.VMEM((S,), jnp.float32),
            pltpu.SemaphoreType.DMA,
            pltpu.SemaphoreType.DMA,
            pltpu.SemaphoreType.DMA,
        ],
    )
    return kfn(vflat, cpat48, ind_l, val_l, ind_k, val_k)


def _tc_tile_call(vertices, center, faces, one_f, one_i):
    BV = 1024
    BF = 2048
    grid = pl.cdiv(V, BV)

    def body(v_ref, c_ref, f_ref, onef_ref, onei_ref, vout_ref, fout_ref):
        vrow = (v_ref[...] + c_ref[...]) * onef_ref[0, 0]
        vout_ref[...] = jnp.broadcast_to(vrow[None], (4, BV, 3))
        fout_ref[...] = jnp.broadcast_to(
            f_ref[...][None] * onei_ref[0, 0], (4, BF, 3))

    return pl.pallas_call(
        body,
        grid=(grid,),
        in_specs=[
            pl.BlockSpec((BV, 3), lambda i: (i, 0)),
            pl.BlockSpec((1, 3), lambda i: (0, 0)),
            pl.BlockSpec((BF, 3), lambda i: (i, 0)),
            pl.BlockSpec(memory_space=pltpu.SMEM),
            pl.BlockSpec(memory_space=pltpu.SMEM),
        ],
        out_specs=[
            pl.BlockSpec((4, BV, 3), lambda i: (0, i, 0)),
            pl.BlockSpec((4, BF, 3), lambda i: (0, i, 0)),
        ],
        out_shape=[
            jax.ShapeDtypeStruct((4, V, 3), jnp.float32),
            jax.ShapeDtypeStruct((4, F, 3), jnp.int32),
        ],
    )(vertices, center, faces, one_f, one_i)


def _tc_reduce_call(parts_flat):
    def body(p_ref, out_ref):
        def comp(j):
            return (p_ref[pl.ds(j * VP, VP)] +
                    p_ref[pl.ds((6 + j) * VP, VP)])
        eps = jnp.float32(1e-12)
        lx, ly, lz = comp(0) + eps, comp(1) + eps, comp(2) + eps
        norm = jnp.sqrt(lx * lx + ly * ly + lz * lz)
        kx, ky, kz = comp(3), comp(4), comp(5)
        ksq = kx * kx + ky * ky + kz * kz
        out_ref[0, 0] = jnp.sum(norm) / jnp.float32(V)
        out_ref[0, 1] = jnp.sum(ksq) / jnp.float32(V)

    return pl.pallas_call(
        body,
        out_specs=pl.BlockSpec(memory_space=pltpu.SMEM),
        out_shape=jax.ShapeDtypeStruct((1, 2), jnp.float32),
    )(parts_flat)


def kernel(vertices, center, faces, L_indices, L_values, K_indices, K_values,
           total_num):
    one_i = jnp.asarray(total_num, dtype=jnp.int32) // 4
    one_f = one_i.astype(jnp.float32)
    one_i_s = jnp.reshape(one_i, (1, 1))
    one_f_s = jnp.reshape(one_f, (1, 1))

    vflat = vertices.reshape(3 * V)
    c3 = center.reshape(3)
    cpat48 = jnp.tile(c3, LANES)

    parts = _sc_spmv_call(vflat, cpat48, L_indices.reshape(2 * NNZ), L_values,
                          K_indices.reshape(2 * NNZ), K_values)
    verts_out, faces_out = _tc_tile_call(vertices, center, faces,
                                         one_f_s, one_i_s)
    losses = _tc_reduce_call(parts)

    laplacian_loss = losses[0, 0]
    hexagon_loss = losses[0, 1]
    zero = jnp.float32(0.0)
    return (verts_out, faces_out, laplacian_loss, hexagon_loss, zero, zero)


# speedup vs baseline: 3.0942x; 1.9496x over previous
import jax
import jax.numpy as jnp
from jax import lax
from jax.experimental import pallas as pl
from jax.experimental.pallas import tpu as pltpu
from jax.experimental.pallas import tpu_sc as plsc

V = 100000
F = 200000
NNZ = 700000

NC = 2
NS = 16
NW = NC * NS
LANES = 16

S = 2736
NSUB = 8
Q = NSUB * S
OVERLAP = NW * Q - NNZ

VP = 100096
CHK = VP // NS


def _sc_spmv_call(vflat, cpat48, ind_l, val_l, ind_k, val_k):
    mesh = plsc.VectorSubcoreMesh(core_axis_name="c", subcore_axis_name="s",
                                  num_cores=NC, num_subcores=NS)

    def body(vflat_hbm, cpat_hbm, iL_hbm, vL_hbm, iK_hbm, vK_hbm,
             out_hbm, vc3_hbm,
             aLx, aLy, aLz, aKx, aKy, aKz, sv3,
             vbuf, xb, cp_v,
             col_v, row_v, val_v, ic0, ic1, ic2, gx, gy, gz,
             sem0, sem1, sem2):
        cid = lax.axis_index("c")
        sid = lax.axis_index("s")

        def zero_body(i, _):
            xb[pl.ds(i * LANES, LANES)] = jnp.zeros((LANES,), jnp.float32)
            return 0
        lax.fori_loop(0, CHK // LANES, zero_body, 0)
        for acc in (aLx, aLy, aLz, aKx, aKy, aKz):
            pltpu.sync_copy(xb, acc.at[pl.ds(sid * CHK, CHK)])

        base0 = jnp.minimum(sid * CHK, V - CHK)
        pltpu.sync_copy(vflat_hbm.at[pl.ds(base0 * 3, 3 * CHK)], vbuf)
        pltpu.sync_copy(cpat_hbm, cp_v)
        q0 = cp_v[pl.ds(0, LANES)]
        q1 = cp_v[pl.ds(LANES, LANES)]
        q2 = cp_v[pl.ds(2 * LANES, LANES)]

        def cadd_body(g, _):
            b = g * (3 * LANES)
            vbuf[pl.ds(b, LANES)] = vbuf[pl.ds(b, LANES)] + q0
            vbuf[pl.ds(b + LANES, LANES)] = (
                vbuf[pl.ds(b + LANES, LANES)] + q1)
            vbuf[pl.ds(b + 2 * LANES, LANES)] = (
                vbuf[pl.ds(b + 2 * LANES, LANES)] + q2)
            return 0
        lax.fori_loop(0, CHK // LANES, cadd_body, 0)
        pltpu.sync_copy(vbuf, sv3.at[pl.ds(base0 * 3, 3 * CHK)])
        @pl.when(cid == 0)
        def _export_vc():
            pltpu.sync_copy(vbuf, vc3_hbm.at[pl.ds(base0 * 3, 3 * CHK)])
        plsc.subcore_barrier()

        wid = cid * NS + sid
        is_last = wid == NW - 1

        def do_matrix(i_hbm, v_hbm, ax, ay, az):
            for j in range(NSUB):
                base = wid * Q + j * S
                if j == NSUB - 1:
                    base = jnp.where(is_last, NNZ - S, base)
                pltpu.sync_copy(i_hbm.at[pl.ds(base, S)], row_v)
                pltpu.sync_copy(i_hbm.at[pl.ds(NNZ + base, S)], col_v)
                pltpu.sync_copy(v_hbm.at[pl.ds(base, S)], val_v)
                if j == NSUB - 1:
                    @pl.when(is_last)
                    def _mask_tail():
                        def zv(i, _):
                            val_v[pl.ds(i * LANES, LANES)] = (
                                jnp.zeros((LANES,), jnp.float32))
                            return 0
                        lax.fori_loop(0, OVERLAP // LANES, zv, 0)

                def idx_body(i, _):
                    sl = pl.ds(i * LANES, LANES)
                    c3 = col_v[sl] * 3
                    ic0[sl] = c3
                    ic1[sl] = c3 + 1
                    ic2[sl] = c3 + 2
                    return 0
                lax.fori_loop(0, S // LANES, idx_body, 0)
                d0 = pltpu.async_copy(sv3.at[ic0], gx, sem0)
                d1 = pltpu.async_copy(sv3.at[ic1], gy, sem1)
                d2 = pltpu.async_copy(sv3.at[ic2], gz, sem2)
                d0.wait()
                d1.wait()
                d2.wait()

                def mul_body(i, _):
                    sl = pl.ds(i * LANES, LANES)
                    w = val_v[sl]
                    gx[sl] = gx[sl] * w
                    gy[sl] = gy[sl] * w
                    gz[sl] = gz[sl] * w
                    return 0
                lax.fori_loop(0, S // LANES, mul_body, 0)

                pltpu.sync_copy(gx, ax.at[row_v], add=True)
                pltpu.sync_copy(gy, ay.at[row_v], add=True)
                pltpu.sync_copy(gz, az.at[row_v], add=True)

        do_matrix(iL_hbm, vL_hbm, aLx, aLy, aLz)
        do_matrix(iK_hbm, vK_hbm, aKx, aKy, aKz)

        plsc.subcore_barrier()

        for j, acc in enumerate((aLx, aLy, aLz, aKx, aKy, aKz)):
            off = (cid * 6 + j) * VP + sid * CHK
            pltpu.sync_copy(acc.at[pl.ds(sid * CHK, CHK)], xb)
            pltpu.sync_copy(xb, out_hbm.at[pl.ds(off, CHK)])

    kfn = pl.kernel(
        body,
        out_type=[jax.ShapeDtypeStruct((NC * 6 * VP,), jnp.float32),
                  jax.ShapeDtypeStruct((3 * V,), jnp.float32)],
        mesh=mesh,
        scratch_types=[
            pltpu.VMEM_SHARED((VP,), jnp.float32),
            pltpu.VMEM_SHARED((VP,), jnp.float32),
            pltpu.VMEM_SHARED((VP,), jnp.float32),
            pltpu.VMEM_SHARED((VP,), jnp.float32),
            pltpu.VMEM_SHARED((VP,), jnp.float32),
            pltpu.VMEM_SHARED((VP,), jnp.float32),
            pltpu.VMEM_SHARED((3 * VP,), jnp.float32),
            pltpu.VMEM((3 * CHK,), jnp.float32),
            pltpu.VMEM((CHK,), jnp.float32),
            pltpu.VMEM((3 * LANES,), jnp.float32),
            pltpu.VMEM((S,), jnp.int32),
            pltpu.VMEM((S,), jnp.int32),
            pltpu.VMEM((S,), jnp.float32),
            pltpu.VMEM((S,), jnp.int32),
            pltpu.VMEM((S,), jnp.int32),
            pltpu.VMEM((S,), jnp.int32),
            pltpu.VMEM((S,), jnp.float32),
            pltpu.VMEM((S,), jnp.float32),
            pltpu.VMEM((S,), jnp.float32),
            pltpu.SemaphoreType.DMA,
            pltpu.SemaphoreType.DMA,
            pltpu.SemaphoreType.DMA,
        ],
    )
    return kfn(vflat, cpat48, ind_l, val_l, ind_k, val_k)


def _tc_reduce_call(parts_flat):
    def body(p_ref, out_ref):
        def comp(j):
            return (p_ref[pl.ds(j * VP, VP)] +
                    p_ref[pl.ds((6 + j) * VP, VP)])
        eps = jnp.float32(1e-12)
        lx, ly, lz = comp(0) + eps, comp(1) + eps, comp(2) + eps
        norm = jnp.sqrt(lx * lx + ly * ly + lz * lz)
        kx, ky, kz = comp(3), comp(4), comp(5)
        ksq = kx * kx + ky * ky + kz * kz
        out_ref[0, 0] = jnp.sum(norm) / jnp.float32(V)
        out_ref[0, 1] = jnp.sum(ksq) / jnp.float32(V)

    return pl.pallas_call(
        body,
        out_specs=pl.BlockSpec(memory_space=pltpu.SMEM),
        out_shape=jax.ShapeDtypeStruct((1, 2), jnp.float32),
    )(parts_flat)


def kernel(vertices, center, faces, L_indices, L_values, K_indices, K_values,
           total_num):
    one_i = jnp.asarray(total_num, dtype=jnp.int32) // 4
    one_f = one_i.astype(jnp.float32)

    vflat = vertices.reshape(3 * V)
    c3 = center.reshape(3)
    cpat48 = jnp.tile(c3, LANES)

    parts, vc3 = _sc_spmv_call(vflat, cpat48, L_indices.reshape(2 * NNZ),
                               L_values, K_indices.reshape(2 * NNZ), K_values)
    losses = _tc_reduce_call(parts)

    vc = vc3.reshape(1, V, 3)
    verts_out = jnp.tile(vc, (4, 1, 1)) * one_f
    faces_out = jnp.tile(faces[None], (4, 1, 1)) * one_i

    laplacian_loss = losses[0, 0]
    hexagon_loss = losses[0, 1]
    zero = jnp.float32(0.0)
    return (verts_out, faces_out, laplacian_loss, hexagon_loss, zero, zero)


# speedup vs baseline: 3.4062x; 1.1008x over previous
import jax
import jax.numpy as jnp
from jax import lax
from jax.experimental import pallas as pl
from jax.experimental.pallas import tpu as pltpu
from jax.experimental.pallas import tpu_sc as plsc

V = 100000
F = 200000
NNZ = 700000

NC = 2
NS = 16
NW = NC * NS
LANES = 16

S = 2736
NSUB = 8
Q = NSUB * S
OVERLAP = NW * Q - NNZ

VP = 100096
CHK = VP // NS


def _sc_spmv_call(vflat, cpat48, ind_l, val_l, ind_k, val_k):
    mesh = plsc.VectorSubcoreMesh(core_axis_name="c", subcore_axis_name="s",
                                  num_cores=NC, num_subcores=NS)

    def body(vflat_hbm, cpat_hbm, iL_hbm, vL_hbm, iK_hbm, vK_hbm,
             out_hbm, vc3_hbm,
             aLx, aLy, aLz, aKx, aKy, aKz, sv3,
             vbuf, xb, cp_v,
             col_v, row_v, val_v, col_w, row_w, val_w,
             ic0, ic1, ic2, gx, gy, gz,
             sem0, sem1, sem2, lsem0, lsem1):
        cid = lax.axis_index("c")
        sid = lax.axis_index("s")

        def zero_body(i, _):
            xb[pl.ds(i * LANES, LANES)] = jnp.zeros((LANES,), jnp.float32)
            return 0
        lax.fori_loop(0, CHK // LANES, zero_body, 0)
        for acc in (aLx, aLy, aLz, aKx, aKy, aKz):
            pltpu.sync_copy(xb, acc.at[pl.ds(sid * CHK, CHK)])

        base0 = jnp.minimum(sid * CHK, V - CHK)
        pltpu.sync_copy(vflat_hbm.at[pl.ds(base0 * 3, 3 * CHK)], vbuf)
        pltpu.sync_copy(cpat_hbm, cp_v)
        q0 = cp_v[pl.ds(0, LANES)]
        q1 = cp_v[pl.ds(LANES, LANES)]
        q2 = cp_v[pl.ds(2 * LANES, LANES)]

        def cadd_body(g, _):
            b = g * (3 * LANES)
            vbuf[pl.ds(b, LANES)] = vbuf[pl.ds(b, LANES)] + q0
            vbuf[pl.ds(b + LANES, LANES)] = (
                vbuf[pl.ds(b + LANES, LANES)] + q1)
            vbuf[pl.ds(b + 2 * LANES, LANES)] = (
                vbuf[pl.ds(b + 2 * LANES, LANES)] + q2)
            return 0
        lax.fori_loop(0, CHK // LANES, cadd_body, 0)
        pltpu.sync_copy(vbuf, sv3.at[pl.ds(base0 * 3, 3 * CHK)])
        @pl.when(cid == 0)
        def _export_vc():
            pltpu.sync_copy(vbuf, vc3_hbm.at[pl.ds(base0 * 3, 3 * CHK)])
        plsc.subcore_barrier()

        wid = cid * NS + sid
        is_last = wid == NW - 1

        chunks = []
        for m, (i_hbm, v_hbm, accs) in enumerate(
                ((iL_hbm, vL_hbm, (aLx, aLy, aLz)),
                 (iK_hbm, vK_hbm, (aKx, aKy, aKz)))):
            for j in range(NSUB):
                base = wid * Q + j * S
                if j == NSUB - 1:
                    base = jnp.where(is_last, NNZ - S, base)
                chunks.append((i_hbm, v_hbm, accs, base, j == NSUB - 1))

        rows = (row_v, row_w)
        cols = (col_v, col_w)
        vals = (val_v, val_w)
        lsems = (lsem0, lsem1)

        def fire_loads(k):
            i_hbm, v_hbm, _, base, _ = chunks[k]
            b = k % 2
            return (
                pltpu.async_copy(i_hbm.at[pl.ds(base, S)], rows[b], lsems[b]),
                pltpu.async_copy(i_hbm.at[pl.ds(NNZ + base, S)], cols[b],
                                 lsems[b]),
                pltpu.async_copy(v_hbm.at[pl.ds(base, S)], vals[b], lsems[b]),
            )

        pending = fire_loads(0)
        for k in range(len(chunks)):
            _, _, (ax, ay, az), _, tail = chunks[k]
            b = k % 2
            for d in pending:
                d.wait()
            if k + 1 < len(chunks):
                pending = fire_loads(k + 1)
            row_b, col_b, val_b = rows[b], cols[b], vals[b]
            if tail:
                @pl.when(is_last)
                def _mask_tail():
                    def zv(i, _):
                        val_b[pl.ds(i * LANES, LANES)] = (
                            jnp.zeros((LANES,), jnp.float32))
                        return 0
                    lax.fori_loop(0, OVERLAP // LANES, zv, 0)

            def idx_body(i, _):
                sl = pl.ds(i * LANES, LANES)
                c3 = col_b[sl] * 3
                ic0[sl] = c3
                ic1[sl] = c3 + 1
                ic2[sl] = c3 + 2
                return 0
            lax.fori_loop(0, S // LANES, idx_body, 0)
            d0 = pltpu.async_copy(sv3.at[ic0], gx, sem0)
            d1 = pltpu.async_copy(sv3.at[ic1], gy, sem1)
            d2 = pltpu.async_copy(sv3.at[ic2], gz, sem2)
            d0.wait()
            d1.wait()
            d2.wait()

            def mul_body(i, _):
                sl = pl.ds(i * LANES, LANES)
                w = val_b[sl]
                gx[sl] = gx[sl] * w
                gy[sl] = gy[sl] * w
                gz[sl] = gz[sl] * w
                return 0
            lax.fori_loop(0, S // LANES, mul_body, 0)

            pltpu.sync_copy(gx, ax.at[row_b], add=True)
            pltpu.sync_copy(gy, ay.at[row_b], add=True)
            pltpu.sync_copy(gz, az.at[row_b], add=True)

        plsc.subcore_barrier()

        for j, acc in enumerate((aLx, aLy, aLz, aKx, aKy, aKz)):
            off = (cid * 6 + j) * VP + sid * CHK
            pltpu.sync_copy(acc.at[pl.ds(sid * CHK, CHK)], xb)
            pltpu.sync_copy(xb, out_hbm.at[pl.ds(off, CHK)])

    kfn = pl.kernel(
        body,
        out_type=[jax.ShapeDtypeStruct((NC * 6 * VP,), jnp.float32),
                  jax.ShapeDtypeStruct((3 * V,), jnp.float32)],
        mesh=mesh,
        scratch_types=[
            pltpu.VMEM_SHARED((VP,), jnp.float32),
            pltpu.VMEM_SHARED((VP,), jnp.float32),
            pltpu.VMEM_SHARED((VP,), jnp.float32),
            pltpu.VMEM_SHARED((VP,), jnp.float32),
            pltpu.VMEM_SHARED((VP,), jnp.float32),
            pltpu.VMEM_SHARED((VP,), jnp.float32),
            pltpu.VMEM_SHARED((3 * VP,), jnp.float32),
            pltpu.VMEM((3 * CHK,), jnp.float32),
            pltpu.VMEM((CHK,), jnp.float32),
            pltpu.VMEM((3 * LANES,), jnp.float32),
            pltpu.VMEM((S,), jnp.int32),
            pltpu.VMEM((S,), jnp.int32),
            pltpu.VMEM((S,), jnp.float32),
            pltpu.VMEM((S,), jnp.int32),
            pltpu.VMEM((S,), jnp.int32),
            pltpu.VMEM((S,), jnp.float32),
            pltpu.VMEM((S,), jnp.int32),
            pltpu.VMEM((S,), jnp.int32),
            pltpu.VMEM((S,), jnp.int32),
            pltpu.VMEM((S,), jnp.float32),
            pltpu.VMEM((S,), jnp.float32),
            pltpu.VMEM((S,), jnp.float32),
            pltpu.SemaphoreType.DMA,
            pltpu.SemaphoreType.DMA,
            pltpu.SemaphoreType.DMA,
            pltpu.SemaphoreType.DMA,
            pltpu.SemaphoreType.DMA,
        ],
    )
    return kfn(vflat, cpat48, ind_l, val_l, ind_k, val_k)


def _tc_reduce_call(parts_flat):
    def body(p_ref, out_ref):
        def comp(j):
            return (p_ref[pl.ds(j * VP, VP)] +
                    p_ref[pl.ds((6 + j) * VP, VP)])
        eps = jnp.float32(1e-12)
        lx, ly, lz = comp(0) + eps, comp(1) + eps, comp(2) + eps
        norm = jnp.sqrt(lx * lx + ly * ly + lz * lz)
        kx, ky, kz = comp(3), comp(4), comp(5)
        ksq = kx * kx + ky * ky + kz * kz
        out_ref[0, 0] = jnp.sum(norm) / jnp.float32(V)
        out_ref[0, 1] = jnp.sum(ksq) / jnp.float32(V)

    return pl.pallas_call(
        body,
        out_specs=pl.BlockSpec(memory_space=pltpu.SMEM),
        out_shape=jax.ShapeDtypeStruct((1, 2), jnp.float32),
    )(parts_flat)


def kernel(vertices, center, faces, L_indices, L_values, K_indices, K_values,
           total_num):
    one_i = jnp.asarray(total_num, dtype=jnp.int32) // 4
    one_f = one_i.astype(jnp.float32)

    vflat = vertices.reshape(3 * V)
    c3 = center.reshape(3)
    cpat48 = jnp.tile(c3, LANES)

    parts, vc3 = _sc_spmv_call(vflat, cpat48, L_indices.reshape(2 * NNZ),
                               L_values, K_indices.reshape(2 * NNZ), K_values)
    losses = _tc_reduce_call(parts)

    vc = vc3.reshape(1, V, 3)
    verts_out = jnp.tile(vc, (4, 1, 1)) * one_f
    faces_out = jnp.tile(faces[None], (4, 1, 1)) * one_i

    laplacian_loss = losses[0, 0]
    hexagon_loss = losses[0, 1]
    zero = jnp.float32(0.0)
    return (verts_out, faces_out, laplacian_loss, hexagon_loss, zero, zero)


# speedup vs baseline: 3.4737x; 1.0198x over previous
import jax
import jax.numpy as jnp
from jax import lax
from jax.experimental import pallas as pl
from jax.experimental.pallas import tpu as pltpu
from jax.experimental.pallas import tpu_sc as plsc

V = 100000
F = 200000
NNZ = 700000

NC = 2
NS = 16
NW = NC * NS
LANES = 16

S = 2736
NSUB = 8
Q = NSUB * S
OVERLAP = NW * Q - NNZ

VP = 100096
CHK = VP // NS


def _sc_spmv_call(vflat, cpat48, ind_l, val_l, ind_k, val_k):
    mesh = plsc.VectorSubcoreMesh(core_axis_name="c", subcore_axis_name="s",
                                  num_cores=NC, num_subcores=NS)

    def body(vflat_hbm, cpat_hbm, iL_hbm, vL_hbm, iK_hbm, vK_hbm,
             out_hbm, vc3_hbm,
             aLx, aLy, aLz, aKx, aKy, aKz, sv3,
             vbuf, xb, cp_v,
             col_v, row_v, val_v, col_w, row_w, val_w,
             ic0, ic1, ic2, jc0, jc1, jc2, gx, gy, gz, hx, hy, hz,
             sem0, sem1, sem2, lsem0, lsem1, ssem0, ssem1):
        cid = lax.axis_index("c")
        sid = lax.axis_index("s")

        def zero_body(i, _):
            xb[pl.ds(i * LANES, LANES)] = jnp.zeros((LANES,), jnp.float32)
            return 0
        lax.fori_loop(0, CHK // LANES, zero_body, 0)
        for acc in (aLx, aLy, aLz, aKx, aKy, aKz):
            pltpu.sync_copy(xb, acc.at[pl.ds(sid * CHK, CHK)])

        base0 = jnp.minimum(sid * CHK, V - CHK)
        pltpu.sync_copy(cpat_hbm, cp_v)
        q0 = cp_v[pl.ds(0, LANES)]
        q1 = cp_v[pl.ds(LANES, LANES)]
        q2 = cp_v[pl.ds(2 * LANES, LANES)]

        for off, ln in ((0, 9408), (9408, 9360)):
            pltpu.sync_copy(vflat_hbm.at[pl.ds(base0 * 3 + off, ln)],
                            vbuf.at[pl.ds(0, ln)])

            def cadd_body(g, _):
                b = g * (3 * LANES)
                vbuf[pl.ds(b, LANES)] = vbuf[pl.ds(b, LANES)] + q0
                vbuf[pl.ds(b + LANES, LANES)] = (
                    vbuf[pl.ds(b + LANES, LANES)] + q1)
                vbuf[pl.ds(b + 2 * LANES, LANES)] = (
                    vbuf[pl.ds(b + 2 * LANES, LANES)] + q2)
                return 0
            lax.fori_loop(0, ln // (3 * LANES), cadd_body, 0)
            pltpu.sync_copy(vbuf.at[pl.ds(0, ln)],
                            sv3.at[pl.ds(base0 * 3 + off, ln)])
            @pl.when(cid == 0)
            def _export_vc():
                pltpu.sync_copy(vbuf.at[pl.ds(0, ln)],
                                vc3_hbm.at[pl.ds(base0 * 3 + off, ln)])
        plsc.subcore_barrier()

        wid = cid * NS + sid
        is_last = wid == NW - 1

        chunks = []
        for m, (i_hbm, v_hbm, accs) in enumerate(
                ((iL_hbm, vL_hbm, (aLx, aLy, aLz)),
                 (iK_hbm, vK_hbm, (aKx, aKy, aKz)))):
            for j in range(NSUB):
                base = wid * Q + j * S
                if j == NSUB - 1:
                    base = jnp.where(is_last, NNZ - S, base)
                chunks.append((i_hbm, v_hbm, accs, base, j == NSUB - 1))

        rows = (row_v, row_w)
        cols = (col_v, col_w)
        vals = (val_v, val_w)
        lsems = (lsem0, lsem1)
        ics = ((ic0, ic1, ic2), (jc0, jc1, jc2))
        gs = ((gx, gy, gz), (hx, hy, hz))
        ssems = (ssem0, ssem1)

        def fire_loads(k):
            i_hbm, v_hbm, _, base, _ = chunks[k]
            b = k % 2
            return (
                pltpu.async_copy(i_hbm.at[pl.ds(base, S)], rows[b], lsems[b]),
                pltpu.async_copy(i_hbm.at[pl.ds(NNZ + base, S)], cols[b],
                                 lsems[b]),
                pltpu.async_copy(v_hbm.at[pl.ds(base, S)], vals[b], lsems[b]),
            )

        pending = fire_loads(0)
        pend_scat = [None, None]
        for k in range(len(chunks)):
            _, _, (ax, ay, az), _, tail = chunks[k]
            b = k % 2
            for d in pending:
                d.wait()
            row_b, col_b, val_b = rows[b], cols[b], vals[b]
            i0, i1, i2 = ics[b]
            g0, g1, g2 = gs[b]
            if tail:
                @pl.when(is_last)
                def _mask_tail():
                    def zv(i, _):
                        val_b[pl.ds(i * LANES, LANES)] = (
                            jnp.zeros((LANES,), jnp.float32))
                        return 0
                    lax.fori_loop(0, OVERLAP // LANES, zv, 0)

            def idx_body(i, _):
                sl = pl.ds(i * LANES, LANES)
                c3 = col_b[sl] * 3
                i0[sl] = c3
                i1[sl] = c3 + 1
                i2[sl] = c3 + 2
                return 0
            lax.fori_loop(0, S // LANES, idx_body, 0)
            if pend_scat[b] is not None:
                for d in pend_scat[b]:
                    d.wait()
                pend_scat[b] = None
            d0 = pltpu.async_copy(sv3.at[i0], g0, sem0)
            d1 = pltpu.async_copy(sv3.at[i1], g1, sem1)
            d2 = pltpu.async_copy(sv3.at[i2], g2, sem2)
            if k + 1 < len(chunks):
                if pend_scat[(k + 1) % 2] is not None:
                    for d in pend_scat[(k + 1) % 2]:
                        d.wait()
                    pend_scat[(k + 1) % 2] = None
                pending = fire_loads(k + 1)
            d0.wait()
            d1.wait()
            d2.wait()

            def mul_body(i, _):
                sl = pl.ds(i * LANES, LANES)
                w = val_b[sl]
                g0[sl] = g0[sl] * w
                g1[sl] = g1[sl] * w
                g2[sl] = g2[sl] * w
                return 0
            lax.fori_loop(0, S // LANES, mul_body, 0)

            pend_scat[b] = (
                pltpu.async_copy(g0, ax.at[row_b], ssems[b], add=True),
                pltpu.async_copy(g1, ay.at[row_b], ssems[b], add=True),
                pltpu.async_copy(g2, az.at[row_b], ssems[b], add=True),
            )

        for ds_ in pend_scat:
            if ds_ is not None:
                for d in ds_:
                    d.wait()
        plsc.subcore_barrier()

        for j, acc in enumerate((aLx, aLy, aLz, aKx, aKy, aKz)):
            off = (cid * 6 + j) * VP + sid * CHK
            pltpu.sync_copy(acc.at[pl.ds(sid * CHK, CHK)], xb)
            pltpu.sync_copy(xb, out_hbm.at[pl.ds(off, CHK)])

    kfn = pl.kernel(
        body,
        out_type=[jax.ShapeDtypeStruct((NC * 6 * VP,), jnp.float32),
                  jax.ShapeDtypeStruct((3 * V,), jnp.float32)],
        mesh=mesh,
        scratch_types=[
            pltpu.VMEM_SHARED((VP,), jnp.float32),
            pltpu.VMEM_SHARED((VP,), jnp.float32),
            pltpu.VMEM_SHARED((VP,), jnp.float32),
            pltpu.VMEM_SHARED((VP,), jnp.float32),
            pltpu.VMEM_SHARED((VP,), jnp.float32),
            pltpu.VMEM_SHARED((VP,), jnp.float32),
            pltpu.VMEM_SHARED((3 * VP,), jnp.float32),
            pltpu.VMEM((9408,), jnp.float32),
            pltpu.VMEM((CHK,), jnp.float32),
            pltpu.VMEM((3 * LANES,), jnp.float32),
            pltpu.VMEM((S,), jnp.int32),
            pltpu.VMEM((S,), jnp.int32),
            pltpu.VMEM((S,), jnp.float32),
            pltpu.VMEM((S,), jnp.int32),
            pltpu.VMEM((S,), jnp.int32),
            pltpu.VMEM((S,), jnp.float32),
            pltpu.VMEM((S,), jnp.int32),
            pltpu.VMEM((S,), jnp.int32),
            pltpu.VMEM((S,), jnp.int32),
            pltpu.VMEM((S,), jnp.int32),
            pltpu.VMEM((S,), jnp.int32),
            pltpu.VMEM((S,), jnp.int32),
            pltpu.VMEM((S,), jnp.float32),
            pltpu.VMEM((S,), jnp.float32),
            pltpu.VMEM((S,), jnp.float32),
            pltpu.VMEM((S,), jnp.float32),
            pltpu.VMEM((S,), jnp.float32),
            pltpu.VMEM((S,), jnp.float32),
            pltpu.SemaphoreType.DMA,
            pltpu.SemaphoreType.DMA,
            pltpu.SemaphoreType.DMA,
            pltpu.SemaphoreType.DMA,
            pltpu.SemaphoreType.DMA,
            pltpu.SemaphoreType.DMA,
            pltpu.SemaphoreType.DMA,
        ],
    )
    return kfn(vflat, cpat48, ind_l, val_l, ind_k, val_k)


def _tc_reduce_call(parts_flat):
    def body(p_ref, out_ref):
        def comp(j):
            return (p_ref[pl.ds(j * VP, VP)] +
                    p_ref[pl.ds((6 + j) * VP, VP)])
        eps = jnp.float32(1e-12)
        lx, ly, lz = comp(0) + eps, comp(1) + eps, comp(2) + eps
        norm = jnp.sqrt(lx * lx + ly * ly + lz * lz)
        kx, ky, kz = comp(3), comp(4), comp(5)
        ksq = kx * kx + ky * ky + kz * kz
        out_ref[0, 0] = jnp.sum(norm) / jnp.float32(V)
        out_ref[0, 1] = jnp.sum(ksq) / jnp.float32(V)

    return pl.pallas_call(
        body,
        out_specs=pl.BlockSpec(memory_space=pltpu.SMEM),
        out_shape=jax.ShapeDtypeStruct((1, 2), jnp.float32),
    )(parts_flat)


def kernel(vertices, center, faces, L_indices, L_values, K_indices, K_values,
           total_num):
    one_i = jnp.asarray(total_num, dtype=jnp.int32) // 4
    one_f = one_i.astype(jnp.float32)

    vflat = vertices.reshape(3 * V)
    c3 = center.reshape(3)
    cpat48 = jnp.tile(c3, LANES)

    parts, vc3 = _sc_spmv_call(vflat, cpat48, L_indices.reshape(2 * NNZ),
                               L_values, K_indices.reshape(2 * NNZ), K_values)
    losses = _tc_reduce_call(parts)

    vc = vc3.reshape(1, V, 3)
    verts_out = jnp.tile(vc, (4, 1, 1)) * one_f
    faces_out = jnp.tile(faces[None], (4, 1, 1)) * one_i

    laplacian_loss = losses[0, 0]
    hexagon_loss = losses[0, 1]
    zero = jnp.float32(0.0)
    return (verts_out, faces_out, laplacian_loss, hexagon_loss, zero, zero)
